# Initial kernel scaffold; baseline (speedup 1.0000x reference)
import jax
import jax.numpy as jnp
from jax.experimental import pallas as pl


def kernel(x, edge_index, W1, att_src1, att_dst1, b1, W2, att_src2, att_dst2, b2):
    raise NotImplementedError("write your pallas kernel here")


# speedup vs baseline: 1.1551x; 1.1551x over previous
import jax
import jax.numpy as jnp
from jax.experimental import pallas as pl
from jax.experimental.pallas import tpu as pltpu

N = 10000
E = 320000
F_IN = 128
HID = 64
H1 = 8


def _matmul_kernel(x_ref, w_ref, o_ref):
    o_ref[...] = jnp.dot(x_ref[...], w_ref[...],
                         preferred_element_type=jnp.float32)


def _pallas_matmul(x, w, block_m=512):
    m, k = x.shape
    _, n = w.shape
    pad_m = (-m) % block_m
    if pad_m:
        x = jnp.pad(x, ((0, pad_m), (0, 0)))
    mp = x.shape[0]
    out = pl.pallas_call(
        _matmul_kernel,
        grid=(mp // block_m,),
        in_specs=[
            pl.BlockSpec((block_m, k), lambda i: (i, 0)),
            pl.BlockSpec((k, n), lambda i: (0, 0)),
        ],
        out_specs=pl.BlockSpec((block_m, n), lambda i: (i, 0)),
        out_shape=jax.ShapeDtypeStruct((mp, n), jnp.float32),
    )(x, w)
    return out[:m]


def _gat_layer(x, W, a_src, a_dst, b, src, dst, heads, out_ch, concat):
    n = x.shape[0]
    h = _pallas_matmul(x, W).reshape(n, heads, out_ch)
    alpha_src = jnp.sum(h * a_src[None, :, :], axis=-1)
    alpha_dst = jnp.sum(h * a_dst[None, :, :], axis=-1)
    e = alpha_src[src] + alpha_dst[dst]
    e = jax.nn.leaky_relu(e, negative_slope=0.2)
    ex = jnp.exp(e)
    denom = jax.ops.segment_sum(ex, dst, num_segments=n)
    msg = h[src] * ex[:, :, None]
    num = jax.ops.segment_sum(msg, dst, num_segments=n)
    out = num / (denom[:, :, None] + 1e-16)
    if concat:
        out = out.reshape(n, heads * out_ch)
    else:
        out = jnp.mean(out, axis=1)
    return out + b


def kernel(x, edge_index, W1, att_src1, att_dst1, b1, W2, att_src2, att_dst2, b2):
    src = edge_index[0]
    dst = edge_index[1]
    z = _gat_layer(x, W1, att_src1, att_dst1, b1, src, dst, H1, HID, True)
    z = jax.nn.elu(z)
    z = _gat_layer(z, W2, att_src2, att_dst2, b2, src, dst, 1, HID, False)
    return z


# speedup vs baseline: 11.0697x; 9.5831x over previous
import functools

import jax
import jax.numpy as jnp
from jax import lax
from jax.experimental import pallas as pl
from jax.experimental.pallas import tpu as pltpu
from jax.experimental.pallas import tpu_sc as plsc

N = 10000
E = 320000
F_IN = 128
HID = 64
H1 = 8

NP = 10240
EP = 327680
BLK = 128
NBLK = EP // BLK
TPB1 = NBLK // 16
TPB2 = NBLK // 32
RW = 80
NA = 10240
NSL = NA // 16
ZR = 128


def _mm1_kernel(x_ref, w_ref, asrc_ref, adst_ref, h_ref, as_o, ad_o):
    h = jnp.dot(x_ref[...], w_ref[0], preferred_element_type=jnp.float32)
    h_ref[0] = h
    as_o[0, 0] = jnp.sum(h * asrc_ref[0, 0][None, :], axis=1)
    ad_o[0, 0] = jnp.sum(h * adst_ref[0, 0][None, :], axis=1)


def _tc_layer1_pre(xp, W1r, a_src, a_dst):
    bm = 1280
    grid = (H1, NP // bm)
    return pl.pallas_call(
        _mm1_kernel,
        grid=grid,
        in_specs=[
            pl.BlockSpec((bm, F_IN), lambda h, i: (i, 0)),
            pl.BlockSpec((1, F_IN, HID), lambda h, i: (h, 0, 0)),
            pl.BlockSpec((1, 1, HID), lambda h, i: (h, 0, 0)),
            pl.BlockSpec((1, 1, HID), lambda h, i: (h, 0, 0)),
        ],
        out_specs=[
            pl.BlockSpec((1, bm, HID), lambda h, i: (h, i, 0)),
            pl.BlockSpec((1, 1, bm), lambda h, i: (h, 0, i)),
            pl.BlockSpec((1, 1, bm), lambda h, i: (h, 0, i)),
        ],
        out_shape=[
            jax.ShapeDtypeStruct((H1, NP, HID), jnp.float32),
            jax.ShapeDtypeStruct((H1, 1, NP), jnp.float32),
            jax.ShapeDtypeStruct((H1, 1, NP), jnp.float32),
        ],
    )(xp, W1r, a_src.reshape(H1, 1, HID), a_dst.reshape(H1, 1, HID))


def _mm2_kernel(acc_ref, b1_ref, w_ref, asrc_ref, adst_ref,
                h_ref, as_o, ad_o):
    cols = []
    for h in range(H1):
        num = acc_ref[h, :, :HID]
        den = acc_ref[h, :, HID:HID + 1]
        cols.append(num / (den + 1e-16))
    z = jnp.concatenate(cols, axis=1) + b1_ref[0][None, :]
    z = jnp.where(z > 0, z, jnp.exp(jnp.minimum(z, 0.0)) - 1.0)
    h2 = jnp.dot(z, w_ref[...], preferred_element_type=jnp.float32)
    h_ref[...] = h2
    as_o[0, 0] = jnp.sum(h2 * asrc_ref[0][None, :], axis=1)
    ad_o[0, 0] = jnp.sum(h2 * adst_ref[0][None, :], axis=1)


def _tc_layer2_pre(acc1, b1, W2, a_src2, a_dst2):
    bm = 1024
    grid = (NA // bm,)
    return pl.pallas_call(
        _mm2_kernel,
        grid=grid,
        in_specs=[
            pl.BlockSpec((H1, bm, RW), lambda i: (0, i, 0)),
            pl.BlockSpec((1, H1 * HID), lambda i: (0, 0)),
            pl.BlockSpec((H1 * HID, HID), lambda i: (0, 0)),
            pl.BlockSpec((1, HID), lambda i: (0, 0)),
            pl.BlockSpec((1, HID), lambda i: (0, 0)),
        ],
        out_specs=[
            pl.BlockSpec((bm, HID), lambda i: (i, 0)),
            pl.BlockSpec((1, 1, bm), lambda i: (i, 0, 0)),
            pl.BlockSpec((1, 1, bm), lambda i: (i, 0, 0)),
        ],
        out_shape=[
            jax.ShapeDtypeStruct((NA, HID), jnp.float32),
            jax.ShapeDtypeStruct((NA // bm, 1, bm), jnp.float32),
            jax.ShapeDtypeStruct((NA // bm, 1, bm), jnp.float32),
        ],
    )(acc1.reshape(H1, NA, RW), b1.reshape(1, H1 * HID), W2,
      a_src2, a_dst2)


def _final_kernel(p_ref, b2_ref, o_ref):
    num = p_ref[0, :, :HID] + p_ref[1, :, :HID]
    den = p_ref[0, :, HID:HID + 1] + p_ref[1, :, HID:HID + 1]
    o_ref[...] = num / (den + 1e-16) + b2_ref[0][None, :]


def _tc_final(acc2, b2):
    bm = 1000
    return pl.pallas_call(
        _final_kernel,
        grid=(N // bm,),
        in_specs=[
            pl.BlockSpec((2, bm, RW), lambda i: (0, i, 0)),
            pl.BlockSpec((1, HID), lambda i: (0, 0)),
        ],
        out_specs=pl.BlockSpec((bm, HID), lambda i: (i, 0)),
        out_shape=jax.ShapeDtypeStruct((N, HID), jnp.float32),
    )(acc2.reshape(2, NA, RW), b2.reshape(1, HID))


def _edge_pass(src_hbm, dst_hbm, src_v, dst_v, asrc_v, adst_v, idx_v, ex_v,
               grow_v, wrow_v, accum, gsem, h_hbm, n_blocks, head_row_off,
               edge_base):
    def block_body(blk, _):
        eoff = edge_base + blk * BLK
        pltpu.sync_copy(src_hbm.at[pl.ds(eoff, BLK)], src_v)
        pltpu.sync_copy(dst_hbm.at[pl.ds(eoff, BLK)], dst_v)

        for k in range(BLK // 16):
            s16 = src_v[pl.ds(k * 16, 16)]
            idx_v[pl.ds(k * 16, 16)] = s16 + head_row_off
        copy = pltpu.make_async_copy(h_hbm.at[idx_v], grow_v, gsem)
        copy.start()

        for k in range(BLK // 16):
            s16 = src_v[pl.ds(k * 16, 16)]
            d16 = dst_v[pl.ds(k * 16, 16)]
            a_s = plsc.load_gather(asrc_v, [s16])
            a_d = plsc.load_gather(adst_v, [d16])
            e16 = a_s + a_d
            e16 = jnp.where(e16 >= 0.0, e16, 0.2 * e16)
            ex16 = jnp.exp(e16)
            gidx = eoff + k * 16 + lax.iota(jnp.int32, 16)
            ex16 = jnp.where(gidx < E, ex16, 0.0)
            ex_v[pl.ds(k * 16, 16)] = ex16

        copy.wait()

        def w_body(j, _):
            jv = lax.broadcast_in_dim(j, (16,), ())
            exv = plsc.load_gather(ex_v, [jv])
            for k in range(HID // 16):
                wrow_v[j, pl.ds(k * 16, 16)] = (
                    grow_v[j, pl.ds(k * 16, 16)] * exv)
            wrow_v[j, pl.ds(HID, 16)] = exv
            return 0

        lax.fori_loop(0, BLK, w_body, 0)

        pltpu.sync_copy(wrow_v, accum.at[dst_v], add=True)
        return 0

    lax.fori_loop(0, n_blocks, block_body, 0)


def _zero_accum(zero_v, accum, row0):
    for i in range(NSL // ZR):
        pltpu.sync_copy(zero_v, accum.at[pl.ds(row0 + i * ZR, ZR)])


def _drain_accum(bounce_v, accum, out_hbm, row0, out_row0):
    for i in range(NSL // ZR):
        pltpu.sync_copy(accum.at[pl.ds(row0 + i * ZR, ZR)], bounce_v)
        pltpu.sync_copy(bounce_v, out_hbm.at[pl.ds(out_row0 + i * ZR, ZR)])


def _init_zero(zero_v):
    z16 = jnp.zeros((16,), jnp.float32)

    def zbody(j, _):
        for k in range(RW // 16):
            zero_v[j, pl.ds(k * 16, 16)] = z16
        return 0

    lax.fori_loop(0, ZR, zbody, 0)


def _sc_layer1(h_t, asrc, adst, srcb, dstb):
    mesh = plsc.VectorSubcoreMesh(core_axis_name="c", subcore_axis_name="s")

    @functools.partial(
        pl.kernel,
        mesh=mesh,
        compiler_params=pltpu.CompilerParams(needs_layout_passes=False, use_tc_tiling_on_sc=False),
        out_type=jax.ShapeDtypeStruct((H1 * NA, RW), jnp.float32),
        scratch_types=[
            pltpu.VMEM((BLK,), jnp.int32),
            pltpu.VMEM((BLK,), jnp.int32),
            pltpu.VMEM((NP,), jnp.float32),
            pltpu.VMEM((NP,), jnp.float32),
            pltpu.VMEM((BLK,), jnp.int32),
            pltpu.VMEM((BLK,), jnp.float32),
            pltpu.VMEM((BLK, HID), jnp.float32),
            pltpu.VMEM((BLK, RW), jnp.float32),
            pltpu.VMEM((ZR, RW), jnp.float32),
            pltpu.VMEM((ZR, RW), jnp.float32),
            pltpu.VMEM_SHARED((NA, RW), jnp.float32),
            pltpu.SemaphoreType.DMA,
        ],
    )
    def k(h_hbm, asrc_hbm, adst_hbm, srcb_hbm, dstb_hbm, out_hbm,
          src_v, dst_v, asrc_v, adst_v, idx_v, ex_v, grow_v, wrow_v,
          zero_v, bounce_v, accum, gsem):
        c = lax.axis_index("c")
        s = lax.axis_index("s")

        _init_zero(zero_v)
        row0 = s * NSL

        def head_body(hi, _):
            head = c * 4 + hi
            _zero_accum(zero_v, accum, row0)
            pltpu.sync_copy(asrc_hbm.at[pl.ds(head * NP, NP)], asrc_v)
            pltpu.sync_copy(adst_hbm.at[pl.ds(head * NP, NP)], adst_v)
            plsc.subcore_barrier()
            _edge_pass(srcb_hbm, dstb_hbm, src_v, dst_v, asrc_v, adst_v,
                       idx_v, ex_v, grow_v, wrow_v, accum, gsem, h_hbm,
                       TPB1, head * NP, s * (TPB1 * BLK))
            plsc.subcore_barrier()
            _drain_accum(bounce_v, accum, out_hbm, row0,
                         head * NA + row0)
            plsc.subcore_barrier()
            return 0

        lax.fori_loop(0, 4, head_body, 0)

    return k(h_t.reshape(H1 * NP, HID), asrc.reshape(H1 * NP),
             adst.reshape(H1 * NP), srcb, dstb)


def _sc_layer2(h2, asrc2, adst2, srcb, dstb):
    mesh = plsc.VectorSubcoreMesh(core_axis_name="c", subcore_axis_name="s")

    @functools.partial(
        pl.kernel,
        mesh=mesh,
        compiler_params=pltpu.CompilerParams(needs_layout_passes=False, use_tc_tiling_on_sc=False),
        out_type=jax.ShapeDtypeStruct((2 * NA, RW), jnp.float32),
        scratch_types=[
            pltpu.VMEM((BLK,), jnp.int32),
            pltpu.VMEM((BLK,), jnp.int32),
            pltpu.VMEM((NA,), jnp.float32),
            pltpu.VMEM((NA,), jnp.float32),
            pltpu.VMEM((BLK,), jnp.int32),
            pltpu.VMEM((BLK,), jnp.float32),
            pltpu.VMEM((BLK, HID), jnp.float32),
            pltpu.VMEM((BLK, RW), jnp.float32),
            pltpu.VMEM((ZR, RW), jnp.float32),
            pltpu.VMEM((ZR, RW), jnp.float32),
            pltpu.VMEM_SHARED((NA, RW), jnp.float32),
            pltpu.SemaphoreType.DMA,
        ],
    )
    def k(h_hbm, asrc_hbm, adst_hbm, srcb_hbm, dstb_hbm, out_hbm,
          src_v, dst_v, asrc_v, adst_v, idx_v, ex_v, grow_v, wrow_v,
          zero_v, bounce_v, accum, gsem):
        c = lax.axis_index("c")
        s = lax.axis_index("s")
        w = c * 16 + s

        pltpu.sync_copy(asrc_hbm, asrc_v)
        pltpu.sync_copy(adst_hbm, adst_v)
        _init_zero(zero_v)
        row0 = s * NSL

        _zero_accum(zero_v, accum, row0)
        plsc.subcore_barrier()
        _edge_pass(srcb_hbm, dstb_hbm, src_v, dst_v, asrc_v, adst_v,
                   idx_v, ex_v, grow_v, wrow_v, accum, gsem, h_hbm,
                   TPB2, 0, w * (TPB2 * BLK))
        plsc.subcore_barrier()
        _drain_accum(bounce_v, accum, out_hbm, row0, c * NA + row0)

    return k(h2, asrc2, adst2, srcb, dstb)


def kernel(x, edge_index, W1, att_src1, att_dst1, b1, W2, att_src2,
           att_dst2, b2):
    src = edge_index[0]
    dst = edge_index[1]
    srcb = jnp.pad(src, (0, EP - E))
    dstb = jnp.pad(dst, (0, EP - E))
    xp = jnp.pad(x, ((0, NP - N), (0, 0)))

    h_t, asrc, adst = _tc_layer1_pre(
        xp, W1.reshape(F_IN, H1, HID).transpose(1, 0, 2), att_src1,
        att_dst1)
    acc1 = _sc_layer1(h_t, asrc, adst, srcb, dstb)

    h2, asrc2, adst2 = _tc_layer2_pre(acc1, b1, W2, att_src2, att_dst2)

    acc2 = _sc_layer2(h2, asrc2.reshape(NA), adst2.reshape(NA), srcb, dstb)

    return _tc_final(acc2, b2)


# speedup vs baseline: 12.2670x; 1.1082x over previous
import functools

import jax
import jax.numpy as jnp
from jax import lax
from jax.experimental import pallas as pl
from jax.experimental.pallas import tpu as pltpu
from jax.experimental.pallas import tpu_sc as plsc

N = 10000
E = 320000
F_IN = 128
HID = 64
H1 = 8

NP = 10240
EP = 327680
BLK = 128
NBLK = EP // BLK
TPB1 = NBLK // 16
TPB2 = NBLK // 32
RW = 80
NA = 10240
NSL = NA // 16
ZR = 128


def _mm1_kernel(x_ref, w_ref, asrc_ref, adst_ref, h_ref, as_o, ad_o):
    h = jnp.dot(x_ref[...], w_ref[0], preferred_element_type=jnp.float32)
    h_ref[0] = h
    as_o[0, 0] = jnp.sum(h * asrc_ref[0, 0][None, :], axis=1)
    ad_o[0, 0] = jnp.sum(h * adst_ref[0, 0][None, :], axis=1)


def _tc_layer1_pre(xp, W1r, a_src, a_dst):
    bm = 1280
    grid = (H1, NP // bm)
    return pl.pallas_call(
        _mm1_kernel,
        grid=grid,
        in_specs=[
            pl.BlockSpec((bm, F_IN), lambda h, i: (i, 0)),
            pl.BlockSpec((1, F_IN, HID), lambda h, i: (h, 0, 0)),
            pl.BlockSpec((1, 1, HID), lambda h, i: (h, 0, 0)),
            pl.BlockSpec((1, 1, HID), lambda h, i: (h, 0, 0)),
        ],
        out_specs=[
            pl.BlockSpec((1, bm, HID), lambda h, i: (h, i, 0)),
            pl.BlockSpec((1, 1, bm), lambda h, i: (h, 0, i)),
            pl.BlockSpec((1, 1, bm), lambda h, i: (h, 0, i)),
        ],
        out_shape=[
            jax.ShapeDtypeStruct((H1, NP, HID), jnp.float32),
            jax.ShapeDtypeStruct((H1, 1, NP), jnp.float32),
            jax.ShapeDtypeStruct((H1, 1, NP), jnp.float32),
        ],
    )(xp, W1r, a_src.reshape(H1, 1, HID), a_dst.reshape(H1, 1, HID))


def _mm2_kernel(acc_ref, b1_ref, w_ref, asrc_ref, adst_ref,
                h_ref, as_o, ad_o):
    cols = []
    for h in range(H1):
        num = acc_ref[h, :, :HID]
        den = acc_ref[h, :, HID:HID + 1]
        cols.append(num / (den + 1e-16))
    z = jnp.concatenate(cols, axis=1) + b1_ref[0][None, :]
    z = jnp.where(z > 0, z, jnp.exp(jnp.minimum(z, 0.0)) - 1.0)
    h2 = jnp.dot(z, w_ref[...], preferred_element_type=jnp.float32)
    h_ref[...] = h2
    as_o[0, 0] = jnp.sum(h2 * asrc_ref[0][None, :], axis=1)
    ad_o[0, 0] = jnp.sum(h2 * adst_ref[0][None, :], axis=1)


def _tc_layer2_pre(acc1, b1, W2, a_src2, a_dst2):
    bm = 1024
    grid = (NA // bm,)
    return pl.pallas_call(
        _mm2_kernel,
        grid=grid,
        in_specs=[
            pl.BlockSpec((H1, bm, RW), lambda i: (0, i, 0)),
            pl.BlockSpec((1, H1 * HID), lambda i: (0, 0)),
            pl.BlockSpec((H1 * HID, HID), lambda i: (0, 0)),
            pl.BlockSpec((1, HID), lambda i: (0, 0)),
            pl.BlockSpec((1, HID), lambda i: (0, 0)),
        ],
        out_specs=[
            pl.BlockSpec((bm, HID), lambda i: (i, 0)),
            pl.BlockSpec((1, 1, bm), lambda i: (i, 0, 0)),
            pl.BlockSpec((1, 1, bm), lambda i: (i, 0, 0)),
        ],
        out_shape=[
            jax.ShapeDtypeStruct((NA, HID), jnp.float32),
            jax.ShapeDtypeStruct((NA // bm, 1, bm), jnp.float32),
            jax.ShapeDtypeStruct((NA // bm, 1, bm), jnp.float32),
        ],
    )(acc1.reshape(H1, NA, RW), b1.reshape(1, H1 * HID), W2,
      a_src2, a_dst2)


def _final_kernel(p_ref, b2_ref, o_ref):
    num = p_ref[0, :, :HID] + p_ref[1, :, :HID]
    den = p_ref[0, :, HID:HID + 1] + p_ref[1, :, HID:HID + 1]
    o_ref[...] = num / (den + 1e-16) + b2_ref[0][None, :]


def _tc_final(acc2, b2):
    bm = 1000
    return pl.pallas_call(
        _final_kernel,
        grid=(N // bm,),
        in_specs=[
            pl.BlockSpec((2, bm, RW), lambda i: (0, i, 0)),
            pl.BlockSpec((1, HID), lambda i: (0, 0)),
        ],
        out_specs=pl.BlockSpec((bm, HID), lambda i: (i, 0)),
        out_shape=jax.ShapeDtypeStruct((N, HID), jnp.float32),
    )(acc2.reshape(2, NA, RW), b2.reshape(1, HID))


def _edge_pass(src_hbm, dst_hbm, src_v, dst_v, dstf, idx0, idx1, ex_v,
               grow0, grow1, wrow, accum, gsem0, gsem1, asrc_v, adst_v,
               h_hbm, n_pairs, head_row_off, edge_base):
    def pair_body(i, _):
        eoff = edge_base + i * (2 * BLK)
        pltpu.sync_copy(src_hbm.at[pl.ds(eoff, 2 * BLK)], src_v)
        pltpu.sync_copy(dst_hbm.at[pl.ds(eoff, 2 * BLK)], dst_v)

        for k in range(BLK // 16):
            s16 = src_v[pl.ds(k * 16, 16)]
            idx0[pl.ds(k * 16, 16)] = s16 + head_row_off
        for k in range(BLK // 16):
            s16 = src_v[pl.ds(BLK + k * 16, 16)]
            idx1[pl.ds(k * 16, 16)] = s16 + head_row_off
        g0 = pltpu.make_async_copy(h_hbm.at[idx0], grow0, gsem0)
        g0.start()
        g1 = pltpu.make_async_copy(h_hbm.at[idx1], grow1, gsem1)
        g1.start()

        for half, grow, g in ((0, grow0, g0), (1, grow1, g1)):
            for k in range(BLK // 16):
                s16 = src_v[pl.ds(half * BLK + k * 16, 16)]
                d16 = dst_v[pl.ds(half * BLK + k * 16, 16)]
                dstf[pl.ds(k * 16, 16)] = d16
                a_s = plsc.load_gather(asrc_v, [s16])
                a_d = plsc.load_gather(adst_v, [d16])
                e16 = a_s + a_d
                e16 = jnp.where(e16 >= 0.0, e16, 0.2 * e16)
                ex16 = jnp.exp(e16)
                gidx = eoff + half * BLK + k * 16 + lax.iota(jnp.int32, 16)
                ex16 = jnp.where(gidx < E, ex16, 0.0)
                ex_v[pl.ds(k * 16, 16)] = ex16

            g.wait()

            def w_body(j, _):
                jv = lax.broadcast_in_dim(j, (16,), ())
                exv = plsc.load_gather(ex_v, [jv])
                for k in range(HID // 16):
                    wrow[j, pl.ds(k * 16, 16)] = (
                        grow[j, pl.ds(k * 16, 16)] * exv)
                wrow[j, pl.ds(HID, 16)] = exv
                return 0

            lax.fori_loop(0, BLK, w_body, 0)

            pltpu.sync_copy(wrow, accum.at[dstf], add=True)
        return 0

    lax.fori_loop(0, n_pairs, pair_body, 0)


def _zero_accum(zero_v, accum, row0):
    for i in range(NSL // ZR):
        pltpu.sync_copy(zero_v, accum.at[pl.ds(row0 + i * ZR, ZR)])


def _drain_accum(wrow, accum, out_hbm, row0, out_row0):
    for i in range(NSL // ZR):
        pltpu.sync_copy(accum.at[pl.ds(row0 + i * ZR, ZR)], wrow)
        pltpu.sync_copy(wrow, out_hbm.at[pl.ds(out_row0 + i * ZR, ZR)])


def _init_zero(zero_v):
    z16 = jnp.zeros((16,), jnp.float32)

    def zbody(j, _):
        for k in range(RW // 16):
            zero_v[j, pl.ds(k * 16, 16)] = z16
        return 0

    lax.fori_loop(0, ZR, zbody, 0)


_SC_SCRATCH = [
    pltpu.VMEM((2 * BLK,), jnp.int32),
    pltpu.VMEM((2 * BLK,), jnp.int32),
    pltpu.VMEM((BLK,), jnp.int32),
    pltpu.VMEM((BLK,), jnp.int32),
    pltpu.VMEM((BLK,), jnp.int32),
    pltpu.VMEM((BLK,), jnp.float32),
    pltpu.VMEM((BLK, HID), jnp.float32),
    pltpu.VMEM((BLK, HID), jnp.float32),
    pltpu.VMEM((BLK, RW), jnp.float32),
    pltpu.VMEM((ZR, RW), jnp.float32),
    pltpu.VMEM((NP,), jnp.float32),
    pltpu.VMEM((NP,), jnp.float32),
    pltpu.VMEM_SHARED((NA, RW), jnp.float32),
    pltpu.SemaphoreType.DMA,
    pltpu.SemaphoreType.DMA,
]


def _sc_layer1(h_t, asrc, adst, srcb, dstb):
    mesh = plsc.VectorSubcoreMesh(core_axis_name="c", subcore_axis_name="s")

    @functools.partial(
        pl.kernel,
        mesh=mesh,
        compiler_params=pltpu.CompilerParams(
            needs_layout_passes=False, use_tc_tiling_on_sc=False),
        out_type=jax.ShapeDtypeStruct((H1 * NA, RW), jnp.float32),
        scratch_types=_SC_SCRATCH,
    )
    def k(h_hbm, asrc_hbm, adst_hbm, srcb_hbm, dstb_hbm, out_hbm,
          src_v, dst_v, dstf, idx0, idx1, ex_v, grow0, grow1, wrow,
          zero_v, asrc_v, adst_v, accum, gsem0, gsem1):
        c = lax.axis_index("c")
        s = lax.axis_index("s")
        row0 = s * NSL

        _init_zero(zero_v)

        def head_body(hi, _):
            head = c * 4 + hi
            _zero_accum(zero_v, accum, row0)
            pltpu.sync_copy(asrc_hbm.at[pl.ds(head * NP, NP)], asrc_v)
            pltpu.sync_copy(adst_hbm.at[pl.ds(head * NP, NP)], adst_v)
            plsc.subcore_barrier()
            _edge_pass(srcb_hbm, dstb_hbm, src_v, dst_v, dstf, idx0,
                       idx1, ex_v, grow0, grow1, wrow, accum, gsem0,
                       gsem1, asrc_v, adst_v, h_hbm, TPB1 // 2,
                       head * NP, s * (TPB1 * BLK))
            plsc.subcore_barrier()
            _drain_accum(wrow, accum, out_hbm, row0, head * NA + row0)
            plsc.subcore_barrier()
            return 0

        lax.fori_loop(0, 4, head_body, 0)

    return k(h_t.reshape(H1 * NP, HID), asrc.reshape(H1 * NP),
             adst.reshape(H1 * NP), srcb, dstb)


def _sc_layer2(h2, asrc2, adst2, srcb, dstb):
    mesh = plsc.VectorSubcoreMesh(core_axis_name="c", subcore_axis_name="s")

    @functools.partial(
        pl.kernel,
        mesh=mesh,
        compiler_params=pltpu.CompilerParams(
            needs_layout_passes=False, use_tc_tiling_on_sc=False),
        out_type=jax.ShapeDtypeStruct((2 * NA, RW), jnp.float32),
        scratch_types=_SC_SCRATCH,
    )
    def k(h_hbm, asrc_hbm, adst_hbm, srcb_hbm, dstb_hbm, out_hbm,
          src_v, dst_v, dstf, idx0, idx1, ex_v, grow0, grow1, wrow,
          zero_v, asrc_v, adst_v, accum, gsem0, gsem1):
        c = lax.axis_index("c")
        s = lax.axis_index("s")
        w = c * 16 + s
        row0 = s * NSL

        pltpu.sync_copy(asrc_hbm, asrc_v.at[pl.ds(0, NA)])
        pltpu.sync_copy(adst_hbm, adst_v.at[pl.ds(0, NA)])
        _init_zero(zero_v)
        _zero_accum(zero_v, accum, row0)
        plsc.subcore_barrier()
        _edge_pass(srcb_hbm, dstb_hbm, src_v, dst_v, dstf, idx0, idx1,
                   ex_v, grow0, grow1, wrow, accum, gsem0, gsem1,
                   asrc_v, adst_v, h_hbm, TPB2 // 2, 0,
                   w * (TPB2 * BLK))
        plsc.subcore_barrier()
        _drain_accum(wrow, accum, out_hbm, row0, c * NA + row0)

    return k(h2, asrc2, adst2, srcb, dstb)


def kernel(x, edge_index, W1, att_src1, att_dst1, b1, W2, att_src2,
           att_dst2, b2):
    src = edge_index[0]
    dst = edge_index[1]
    srcb = jnp.pad(src, (0, EP - E))
    dstb = jnp.pad(dst, (0, EP - E))
    xp = jnp.pad(x, ((0, NP - N), (0, 0)))

    h_t, asrc, adst = _tc_layer1_pre(
        xp, W1.reshape(F_IN, H1, HID).transpose(1, 0, 2), att_src1,
        att_dst1)
    acc1 = _sc_layer1(h_t, asrc, adst, srcb, dstb)

    h2, asrc2, adst2 = _tc_layer2_pre(acc1, b1, W2, att_src2, att_dst2)

    acc2 = _sc_layer2(h2, asrc2.reshape(NA), adst2.reshape(NA), srcb, dstb)

    return _tc_final(acc2, b2)


# speedup vs baseline: 12.9696x; 1.0573x over previous
import functools

import jax
import jax.numpy as jnp
from jax import lax
from jax.experimental import pallas as pl
from jax.experimental.pallas import tpu as pltpu
from jax.experimental.pallas import tpu_sc as plsc

N = 10000
E = 320000
F_IN = 128
HID = 64
H1 = 8

NP = 10240
EP = 327680
BLK = 128
NBLK = EP // BLK
TPB1 = NBLK // 16
TPB2 = NBLK // 32
RW = 80
NA = 10240
NSL = NA // 16
ZR = 128


def _mm1_kernel(x_ref, w_ref, asrc_ref, adst_ref, h_ref, as_o, ad_o):
    h = jnp.dot(x_ref[...], w_ref[0], preferred_element_type=jnp.float32)
    h_ref[0] = h
    as_o[0, 0] = jnp.sum(h * asrc_ref[0, 0][None, :], axis=1)
    ad_o[0, 0] = jnp.sum(h * adst_ref[0, 0][None, :], axis=1)


def _tc_layer1_pre(xp, W1r, a_src, a_dst):
    bm = 1280
    grid = (H1, NP // bm)
    return pl.pallas_call(
        _mm1_kernel,
        grid=grid,
        in_specs=[
            pl.BlockSpec((bm, F_IN), lambda h, i: (i, 0)),
            pl.BlockSpec((1, F_IN, HID), lambda h, i: (h, 0, 0)),
            pl.BlockSpec((1, 1, HID), lambda h, i: (h, 0, 0)),
            pl.BlockSpec((1, 1, HID), lambda h, i: (h, 0, 0)),
        ],
        out_specs=[
            pl.BlockSpec((1, bm, HID), lambda h, i: (h, i, 0)),
            pl.BlockSpec((1, 1, bm), lambda h, i: (h, 0, i)),
            pl.BlockSpec((1, 1, bm), lambda h, i: (h, 0, i)),
        ],
        out_shape=[
            jax.ShapeDtypeStruct((H1, NP, HID), jnp.float32),
            jax.ShapeDtypeStruct((H1, 1, NP), jnp.float32),
            jax.ShapeDtypeStruct((H1, 1, NP), jnp.float32),
        ],
    )(xp, W1r, a_src.reshape(H1, 1, HID), a_dst.reshape(H1, 1, HID))


def _mm2_kernel(acc_ref, b1_ref, w_ref, asrc_ref, adst_ref,
                h_ref, as_o, ad_o):
    cols = []
    for h in range(H1):
        num = acc_ref[h, :, :HID]
        den = acc_ref[h, :, HID:HID + 1]
        cols.append(num / (den + 1e-16))
    z = jnp.concatenate(cols, axis=1) + b1_ref[0][None, :]
    z = jnp.where(z > 0, z, jnp.exp(jnp.minimum(z, 0.0)) - 1.0)
    h2 = jnp.dot(z, w_ref[...], preferred_element_type=jnp.float32)
    h_ref[...] = h2
    as_o[0, 0] = jnp.sum(h2 * asrc_ref[0][None, :], axis=1)
    ad_o[0, 0] = jnp.sum(h2 * adst_ref[0][None, :], axis=1)


def _tc_layer2_pre(acc1, b1, W2, a_src2, a_dst2):
    bm = 1024
    grid = (NA // bm,)
    return pl.pallas_call(
        _mm2_kernel,
        grid=grid,
        in_specs=[
            pl.BlockSpec((H1, bm, RW), lambda i: (0, i, 0)),
            pl.BlockSpec((1, H1 * HID), lambda i: (0, 0)),
            pl.BlockSpec((H1 * HID, HID), lambda i: (0, 0)),
            pl.BlockSpec((1, HID), lambda i: (0, 0)),
            pl.BlockSpec((1, HID), lambda i: (0, 0)),
        ],
        out_specs=[
            pl.BlockSpec((bm, HID), lambda i: (i, 0)),
            pl.BlockSpec((1, 1, bm), lambda i: (i, 0, 0)),
            pl.BlockSpec((1, 1, bm), lambda i: (i, 0, 0)),
        ],
        out_shape=[
            jax.ShapeDtypeStruct((NA, HID), jnp.float32),
            jax.ShapeDtypeStruct((NA // bm, 1, bm), jnp.float32),
            jax.ShapeDtypeStruct((NA // bm, 1, bm), jnp.float32),
        ],
    )(acc1.reshape(H1, NA, RW), b1.reshape(1, H1 * HID), W2,
      a_src2, a_dst2)


def _final_kernel(p_ref, b2_ref, o_ref):
    num = p_ref[0, :, :HID] + p_ref[1, :, :HID]
    den = p_ref[0, :, HID:HID + 1] + p_ref[1, :, HID:HID + 1]
    o_ref[...] = num / (den + 1e-16) + b2_ref[0][None, :]


def _tc_final(acc2, b2):
    bm = 1000
    return pl.pallas_call(
        _final_kernel,
        grid=(N // bm,),
        in_specs=[
            pl.BlockSpec((2, bm, RW), lambda i: (0, i, 0)),
            pl.BlockSpec((1, HID), lambda i: (0, 0)),
        ],
        out_specs=pl.BlockSpec((bm, HID), lambda i: (i, 0)),
        out_shape=jax.ShapeDtypeStruct((N, HID), jnp.float32),
    )(acc2.reshape(2, NA, RW), b2.reshape(1, HID))


def _edge_pass(src_hbm, dst_hbm, src_v, dst_v, dstf0, dstf1, idx0, idx1,
               ex_v, grow0, grow1, wrow0, wrow1, accum, gsem0, gsem1,
               ssem, asrc_v, adst_v, h_hbm, n_pairs, head_row_off,
               edge_base):
    def weight(grow, wrow):
        def w_body(j, _):
            jv = lax.broadcast_in_dim(j, (16,), ())
            exv = plsc.load_gather(ex_v, [jv])
            for k in range(HID // 16):
                wrow[j, pl.ds(k * 16, 16)] = (
                    grow[j, pl.ds(k * 16, 16)] * exv)
            wrow[j, pl.ds(HID, 16)] = exv
            return 0

        lax.fori_loop(0, BLK, w_body, 0, unroll=4)

    def ex_block(eoff, half, dstf):
        for k in range(BLK // 16):
            s16 = src_v[pl.ds(half * BLK + k * 16, 16)]
            d16 = dst_v[pl.ds(half * BLK + k * 16, 16)]
            dstf[pl.ds(k * 16, 16)] = d16
            a_s = plsc.load_gather(asrc_v, [s16])
            a_d = plsc.load_gather(adst_v, [d16])
            e16 = a_s + a_d
            e16 = jnp.where(e16 >= 0.0, e16, 0.2 * e16)
            ex16 = jnp.exp(e16)
            gidx = eoff + half * BLK + k * 16 + lax.iota(jnp.int32, 16)
            ex16 = jnp.where(gidx < E, ex16, 0.0)
            ex_v[pl.ds(k * 16, 16)] = ex16

    def pair_body(i, _):
        eoff = edge_base + i * (2 * BLK)
        pltpu.sync_copy(src_hbm.at[pl.ds(eoff, 2 * BLK)], src_v)
        pltpu.sync_copy(dst_hbm.at[pl.ds(eoff, 2 * BLK)], dst_v)

        for k in range(BLK // 16):
            s16 = src_v[pl.ds(k * 16, 16)]
            idx0[pl.ds(k * 16, 16)] = s16 + head_row_off
        for k in range(BLK // 16):
            s16 = src_v[pl.ds(BLK + k * 16, 16)]
            idx1[pl.ds(k * 16, 16)] = s16 + head_row_off
        g0 = pltpu.make_async_copy(h_hbm.at[idx0], grow0, gsem0)
        g0.start()
        g1 = pltpu.make_async_copy(h_hbm.at[idx1], grow1, gsem1)
        g1.start()

        ex_block(eoff, 0, dstf0)
        g0.wait()
        weight(grow0, wrow0)
        sc0 = pltpu.make_async_copy(wrow0, accum.at[dstf0], ssem)
        sc0.start(add=True)

        ex_block(eoff, 1, dstf1)
        g1.wait()
        weight(grow1, wrow1)
        sc0.wait()
        pltpu.sync_copy(wrow1, accum.at[dstf1], add=True)
        return 0

    lax.fori_loop(0, n_pairs, pair_body, 0)


def _zero_accum(wrow0, accum, row0):
    for i in range(NSL // ZR):
        pltpu.sync_copy(wrow0, accum.at[pl.ds(row0 + i * ZR, ZR)])


def _drain_accum(wrow1, accum, out_hbm, row0, out_row0):
    for i in range(NSL // ZR):
        pltpu.sync_copy(accum.at[pl.ds(row0 + i * ZR, ZR)], wrow1)
        pltpu.sync_copy(wrow1, out_hbm.at[pl.ds(out_row0 + i * ZR, ZR)])


def _init_zero(wrow0):
    z16 = jnp.zeros((16,), jnp.float32)

    def zbody(j, _):
        for k in range(RW // 16):
            wrow0[j, pl.ds(k * 16, 16)] = z16
        return 0

    lax.fori_loop(0, ZR, zbody, 0)


_SC_SCRATCH = [
    pltpu.VMEM((2 * BLK,), jnp.int32),
    pltpu.VMEM((2 * BLK,), jnp.int32),
    pltpu.VMEM((BLK,), jnp.int32),
    pltpu.VMEM((BLK,), jnp.int32),
    pltpu.VMEM((BLK,), jnp.int32),
    pltpu.VMEM((BLK,), jnp.int32),
    pltpu.VMEM((BLK,), jnp.float32),
    pltpu.VMEM((BLK, HID), jnp.float32),
    pltpu.VMEM((BLK, HID), jnp.float32),
    pltpu.VMEM((BLK, RW), jnp.float32),
    pltpu.VMEM((BLK, RW), jnp.float32),
    pltpu.VMEM((NP,), jnp.float32),
    pltpu.VMEM((NP,), jnp.float32),
    pltpu.VMEM_SHARED((NA, RW), jnp.float32),
    pltpu.SemaphoreType.DMA,
    pltpu.SemaphoreType.DMA,
    pltpu.SemaphoreType.DMA,
]


def _sc_layer1(h_t, asrc, adst, srcb, dstb):
    mesh = plsc.VectorSubcoreMesh(core_axis_name="c", subcore_axis_name="s")

    @functools.partial(
        pl.kernel,
        mesh=mesh,
        compiler_params=pltpu.CompilerParams(
            needs_layout_passes=False, use_tc_tiling_on_sc=False),
        out_type=jax.ShapeDtypeStruct((H1 * NA, RW), jnp.float32),
        scratch_types=_SC_SCRATCH,
    )
    def k(h_hbm, asrc_hbm, adst_hbm, srcb_hbm, dstb_hbm, out_hbm,
          src_v, dst_v, dstf0, dstf1, idx0, idx1, ex_v, grow0, grow1,
          wrow0, wrow1, asrc_v, adst_v, accum, gsem0, gsem1, ssem):
        c = lax.axis_index("c")
        s = lax.axis_index("s")
        row0 = s * NSL

        def head_body(hi, _):
            head = c * 4 + hi
            _init_zero(wrow0)
            _zero_accum(wrow0, accum, row0)
            pltpu.sync_copy(asrc_hbm.at[pl.ds(head * NP, NP)], asrc_v)
            pltpu.sync_copy(adst_hbm.at[pl.ds(head * NP, NP)], adst_v)
            plsc.subcore_barrier()
            _edge_pass(srcb_hbm, dstb_hbm, src_v, dst_v, dstf0, dstf1,
                       idx0, idx1, ex_v, grow0, grow1, wrow0, wrow1,
                       accum, gsem0, gsem1, ssem, asrc_v, adst_v,
                       h_hbm, TPB1 // 2, head * NP, s * (TPB1 * BLK))
            plsc.subcore_barrier()
            _drain_accum(wrow1, accum, out_hbm, row0, head * NA + row0)
            plsc.subcore_barrier()
            return 0

        lax.fori_loop(0, 4, head_body, 0)

    return k(h_t.reshape(H1 * NP, HID), asrc.reshape(H1 * NP),
             adst.reshape(H1 * NP), srcb, dstb)


def _sc_layer2(h2, asrc2, adst2, srcb, dstb):
    mesh = plsc.VectorSubcoreMesh(core_axis_name="c", subcore_axis_name="s")

    @functools.partial(
        pl.kernel,
        mesh=mesh,
        compiler_params=pltpu.CompilerParams(
            needs_layout_passes=False, use_tc_tiling_on_sc=False),
        out_type=jax.ShapeDtypeStruct((2 * NA, RW), jnp.float32),
        scratch_types=_SC_SCRATCH,
    )
    def k(h_hbm, asrc_hbm, adst_hbm, srcb_hbm, dstb_hbm, out_hbm,
          src_v, dst_v, dstf0, dstf1, idx0, idx1, ex_v, grow0, grow1,
          wrow0, wrow1, asrc_v, adst_v, accum, gsem0, gsem1, ssem):
        c = lax.axis_index("c")
        s = lax.axis_index("s")
        w = c * 16 + s
        row0 = s * NSL

        pltpu.sync_copy(asrc_hbm, asrc_v.at[pl.ds(0, NA)])
        pltpu.sync_copy(adst_hbm, adst_v.at[pl.ds(0, NA)])
        _init_zero(wrow0)
        _zero_accum(wrow0, accum, row0)
        plsc.subcore_barrier()
        _edge_pass(srcb_hbm, dstb_hbm, src_v, dst_v, dstf0, dstf1,
                   idx0, idx1, ex_v, grow0, grow1, wrow0, wrow1,
                   accum, gsem0, gsem1, ssem, asrc_v, adst_v, h_hbm,
                   TPB2 // 2, 0, w * (TPB2 * BLK))
        plsc.subcore_barrier()
        _drain_accum(wrow1, accum, out_hbm, row0, c * NA + row0)

    return k(h2, asrc2, adst2, srcb, dstb)


def kernel(x, edge_index, W1, att_src1, att_dst1, b1, W2, att_src2,
           att_dst2, b2):
    src = edge_index[0]
    dst = edge_index[1]
    srcb = jnp.pad(src, (0, EP - E))
    dstb = jnp.pad(dst, (0, EP - E))
    xp = jnp.pad(x, ((0, NP - N), (0, 0)))

    h_t, asrc, adst = _tc_layer1_pre(
        xp, W1.reshape(F_IN, H1, HID).transpose(1, 0, 2), att_src1,
        att_dst1)
    acc1 = _sc_layer1(h_t, asrc, adst, srcb, dstb)

    h2, asrc2, adst2 = _tc_layer2_pre(acc1, b1, W2, att_src2, att_dst2)

    acc2 = _sc_layer2(h2, asrc2.reshape(NA), adst2.reshape(NA), srcb, dstb)

    return _tc_final(acc2, b2)


# speedup vs baseline: 13.0166x; 1.0036x over previous
import functools

import jax
import jax.numpy as jnp
from jax import lax
from jax.experimental import pallas as pl
from jax.experimental.pallas import tpu as pltpu
from jax.experimental.pallas import tpu_sc as plsc

N = 10000
E = 320000
F_IN = 128
HID = 64
H1 = 8

NP = 10240
EP = 327680
BLK = 128
NBLK = EP // BLK
TPB1 = NBLK // 16
TPB2 = NBLK // 32
RW = 80
NA = 10240
NSL = NA // 16
ZR = 128


def _mm1_kernel(x_ref, w_ref, asrc_ref, adst_ref, h_ref, as_o, ad_o):
    h = jnp.dot(x_ref[...], w_ref[0], preferred_element_type=jnp.float32)
    h_ref[0] = h
    as_o[0, 0] = jnp.sum(h * asrc_ref[0, 0][None, :], axis=1)
    ad_o[0, 0] = jnp.sum(h * adst_ref[0, 0][None, :], axis=1)


def _tc_layer1_pre(xp, W1r, a_src, a_dst):
    bm = 1280
    grid = (H1, NP // bm)
    return pl.pallas_call(
        _mm1_kernel,
        grid=grid,
        in_specs=[
            pl.BlockSpec((bm, F_IN), lambda h, i: (i, 0)),
            pl.BlockSpec((1, F_IN, HID), lambda h, i: (h, 0, 0)),
            pl.BlockSpec((1, 1, HID), lambda h, i: (h, 0, 0)),
            pl.BlockSpec((1, 1, HID), lambda h, i: (h, 0, 0)),
        ],
        out_specs=[
            pl.BlockSpec((1, bm, HID), lambda h, i: (h, i, 0)),
            pl.BlockSpec((1, 1, bm), lambda h, i: (h, 0, i)),
            pl.BlockSpec((1, 1, bm), lambda h, i: (h, 0, i)),
        ],
        out_shape=[
            jax.ShapeDtypeStruct((H1, NP, HID), jnp.float32),
            jax.ShapeDtypeStruct((H1, 1, NP), jnp.float32),
            jax.ShapeDtypeStruct((H1, 1, NP), jnp.float32),
        ],
    )(xp, W1r, a_src.reshape(H1, 1, HID), a_dst.reshape(H1, 1, HID))


def _mm2_kernel(acc_ref, b1_ref, w_ref, asrc_ref, adst_ref,
                h_ref, as_o, ad_o):
    cols = []
    for h in range(H1):
        num = acc_ref[h, :, :HID]
        den = acc_ref[h, :, HID:HID + 1]
        cols.append(num / (den + 1e-16))
    z = jnp.concatenate(cols, axis=1) + b1_ref[0][None, :]
    z = jnp.where(z > 0, z, jnp.exp(jnp.minimum(z, 0.0)) - 1.0)
    h2 = jnp.dot(z, w_ref[...], preferred_element_type=jnp.float32)
    h_ref[...] = h2
    as_o[0, 0] = jnp.sum(h2 * asrc_ref[0][None, :], axis=1)
    ad_o[0, 0] = jnp.sum(h2 * adst_ref[0][None, :], axis=1)


def _tc_layer2_pre(acc1, b1, W2, a_src2, a_dst2):
    bm = 1024
    grid = (NA // bm,)
    return pl.pallas_call(
        _mm2_kernel,
        grid=grid,
        in_specs=[
            pl.BlockSpec((H1, bm, RW), lambda i: (0, i, 0)),
            pl.BlockSpec((1, H1 * HID), lambda i: (0, 0)),
            pl.BlockSpec((H1 * HID, HID), lambda i: (0, 0)),
            pl.BlockSpec((1, HID), lambda i: (0, 0)),
            pl.BlockSpec((1, HID), lambda i: (0, 0)),
        ],
        out_specs=[
            pl.BlockSpec((bm, HID), lambda i: (i, 0)),
            pl.BlockSpec((1, 1, bm), lambda i: (i, 0, 0)),
            pl.BlockSpec((1, 1, bm), lambda i: (i, 0, 0)),
        ],
        out_shape=[
            jax.ShapeDtypeStruct((NA, HID), jnp.float32),
            jax.ShapeDtypeStruct((NA // bm, 1, bm), jnp.float32),
            jax.ShapeDtypeStruct((NA // bm, 1, bm), jnp.float32),
        ],
    )(acc1.reshape(H1, NA, RW), b1.reshape(1, H1 * HID), W2,
      a_src2, a_dst2)


def _final_kernel(p_ref, b2_ref, o_ref):
    num = p_ref[0, :, :HID] + p_ref[1, :, :HID]
    den = p_ref[0, :, HID:HID + 1] + p_ref[1, :, HID:HID + 1]
    o_ref[...] = num / (den + 1e-16) + b2_ref[0][None, :]


def _tc_final(acc2, b2):
    bm = 1000
    return pl.pallas_call(
        _final_kernel,
        grid=(N // bm,),
        in_specs=[
            pl.BlockSpec((2, bm, RW), lambda i: (0, i, 0)),
            pl.BlockSpec((1, HID), lambda i: (0, 0)),
        ],
        out_specs=pl.BlockSpec((bm, HID), lambda i: (i, 0)),
        out_shape=jax.ShapeDtypeStruct((N, HID), jnp.float32),
    )(acc2.reshape(2, NA, RW), b2.reshape(1, HID))


def _edge_pass(src_hbm, dst_hbm, src_v, dst_v, dstf0, dstf1, idx0, idx1,
               ex_v, grow0, grow1, wrow0, wrow1, accum, gsem0, gsem1,
               ssem, asrc_v, adst_v, h_hbm, n_pairs, head_row_off,
               edge_base):
    def weight(grow, wrow):
        def w_body(j, _):
            jv = lax.broadcast_in_dim(j, (16,), ())
            exv = plsc.load_gather(ex_v, [jv])
            for k in range(HID // 16):
                wrow[j, pl.ds(k * 16, 16)] = (
                    grow[j, pl.ds(k * 16, 16)] * exv)
            wrow[j, pl.ds(HID, 16)] = exv
            return 0

        lax.fori_loop(0, BLK, w_body, 0, unroll=8)

    def ex_block(eoff, half, dstf):
        for k in range(BLK // 16):
            s16 = src_v[pl.ds(half * BLK + k * 16, 16)]
            d16 = dst_v[pl.ds(half * BLK + k * 16, 16)]
            dstf[pl.ds(k * 16, 16)] = d16
            a_s = plsc.load_gather(asrc_v, [s16])
            a_d = plsc.load_gather(adst_v, [d16])
            e16 = a_s + a_d
            e16 = jnp.where(e16 >= 0.0, e16, 0.2 * e16)
            ex16 = jnp.exp(e16)
            gidx = eoff + half * BLK + k * 16 + lax.iota(jnp.int32, 16)
            ex16 = jnp.where(gidx < E, ex16, 0.0)
            ex_v[pl.ds(k * 16, 16)] = ex16

    def pair_body(i, _):
        eoff = edge_base + i * (2 * BLK)
        pltpu.sync_copy(src_hbm.at[pl.ds(eoff, 2 * BLK)], src_v)
        pltpu.sync_copy(dst_hbm.at[pl.ds(eoff, 2 * BLK)], dst_v)

        for k in range(BLK // 16):
            s16 = src_v[pl.ds(k * 16, 16)]
            idx0[pl.ds(k * 16, 16)] = s16 + head_row_off
        for k in range(BLK // 16):
            s16 = src_v[pl.ds(BLK + k * 16, 16)]
            idx1[pl.ds(k * 16, 16)] = s16 + head_row_off
        g0 = pltpu.make_async_copy(h_hbm.at[idx0], grow0, gsem0)
        g0.start()
        g1 = pltpu.make_async_copy(h_hbm.at[idx1], grow1, gsem1)
        g1.start()

        ex_block(eoff, 0, dstf0)
        g0.wait()
        weight(grow0, wrow0)
        sc0 = pltpu.make_async_copy(wrow0, accum.at[dstf0], ssem)
        sc0.start(add=True)

        ex_block(eoff, 1, dstf1)
        g1.wait()
        weight(grow1, wrow1)
        sc0.wait()
        pltpu.sync_copy(wrow1, accum.at[dstf1], add=True)
        return 0

    lax.fori_loop(0, n_pairs, pair_body, 0)


def _zero_accum(wrow0, accum, row0):
    for i in range(NSL // ZR):
        pltpu.sync_copy(wrow0, accum.at[pl.ds(row0 + i * ZR, ZR)])


def _drain_accum(wrow1, accum, out_hbm, row0, out_row0):
    for i in range(NSL // ZR):
        pltpu.sync_copy(accum.at[pl.ds(row0 + i * ZR, ZR)], wrow1)
        pltpu.sync_copy(wrow1, out_hbm.at[pl.ds(out_row0 + i * ZR, ZR)])


def _init_zero(wrow0):
    z16 = jnp.zeros((16,), jnp.float32)

    def zbody(j, _):
        for k in range(RW // 16):
            wrow0[j, pl.ds(k * 16, 16)] = z16
        return 0

    lax.fori_loop(0, ZR, zbody, 0)


_SC_SCRATCH = [
    pltpu.VMEM((2 * BLK,), jnp.int32),
    pltpu.VMEM((2 * BLK,), jnp.int32),
    pltpu.VMEM((BLK,), jnp.int32),
    pltpu.VMEM((BLK,), jnp.int32),
    pltpu.VMEM((BLK,), jnp.int32),
    pltpu.VMEM((BLK,), jnp.int32),
    pltpu.VMEM((BLK,), jnp.float32),
    pltpu.VMEM((BLK, HID), jnp.float32),
    pltpu.VMEM((BLK, HID), jnp.float32),
    pltpu.VMEM((BLK, RW), jnp.float32),
    pltpu.VMEM((BLK, RW), jnp.float32),
    pltpu.VMEM((NP,), jnp.float32),
    pltpu.VMEM((NP,), jnp.float32),
    pltpu.VMEM_SHARED((NA, RW), jnp.float32),
    pltpu.SemaphoreType.DMA,
    pltpu.SemaphoreType.DMA,
    pltpu.SemaphoreType.DMA,
]


def _sc_layer1(h_t, asrc, adst, srcb, dstb):
    mesh = plsc.VectorSubcoreMesh(core_axis_name="c", subcore_axis_name="s")

    @functools.partial(
        pl.kernel,
        mesh=mesh,
        compiler_params=pltpu.CompilerParams(
            needs_layout_passes=False, use_tc_tiling_on_sc=False),
        out_type=jax.ShapeDtypeStruct((H1 * NA, RW), jnp.float32),
        scratch_types=_SC_SCRATCH,
    )
    def k(h_hbm, asrc_hbm, adst_hbm, srcb_hbm, dstb_hbm, out_hbm,
          src_v, dst_v, dstf0, dstf1, idx0, idx1, ex_v, grow0, grow1,
          wrow0, wrow1, asrc_v, adst_v, accum, gsem0, gsem1, ssem):
        c = lax.axis_index("c")
        s = lax.axis_index("s")
        row0 = s * NSL

        def head_body(hi, _):
            head = c * 4 + hi
            _init_zero(wrow0)
            _zero_accum(wrow0, accum, row0)
            pltpu.sync_copy(asrc_hbm.at[pl.ds(head * NP, NP)], asrc_v)
            pltpu.sync_copy(adst_hbm.at[pl.ds(head * NP, NP)], adst_v)
            plsc.subcore_barrier()
            _edge_pass(srcb_hbm, dstb_hbm, src_v, dst_v, dstf0, dstf1,
                       idx0, idx1, ex_v, grow0, grow1, wrow0, wrow1,
                       accum, gsem0, gsem1, ssem, asrc_v, adst_v,
                       h_hbm, TPB1 // 2, head * NP, s * (TPB1 * BLK))
            plsc.subcore_barrier()
            _drain_accum(wrow1, accum, out_hbm, row0, head * NA + row0)
            plsc.subcore_barrier()
            return 0

        lax.fori_loop(0, 4, head_body, 0)

    return k(h_t.reshape(H1 * NP, HID), asrc.reshape(H1 * NP),
             adst.reshape(H1 * NP), srcb, dstb)


def _sc_layer2(h2, asrc2, adst2, srcb, dstb):
    mesh = plsc.VectorSubcoreMesh(core_axis_name="c", subcore_axis_name="s")

    @functools.partial(
        pl.kernel,
        mesh=mesh,
        compiler_params=pltpu.CompilerParams(
            needs_layout_passes=False, use_tc_tiling_on_sc=False),
        out_type=jax.ShapeDtypeStruct((2 * NA, RW), jnp.float32),
        scratch_types=_SC_SCRATCH,
    )
    def k(h_hbm, asrc_hbm, adst_hbm, srcb_hbm, dstb_hbm, out_hbm,
          src_v, dst_v, dstf0, dstf1, idx0, idx1, ex_v, grow0, grow1,
          wrow0, wrow1, asrc_v, adst_v, accum, gsem0, gsem1, ssem):
        c = lax.axis_index("c")
        s = lax.axis_index("s")
        w = c * 16 + s
        row0 = s * NSL

        pltpu.sync_copy(asrc_hbm, asrc_v.at[pl.ds(0, NA)])
        pltpu.sync_copy(adst_hbm, adst_v.at[pl.ds(0, NA)])
        _init_zero(wrow0)
        _zero_accum(wrow0, accum, row0)
        plsc.subcore_barrier()
        _edge_pass(srcb_hbm, dstb_hbm, src_v, dst_v, dstf0, dstf1,
                   idx0, idx1, ex_v, grow0, grow1, wrow0, wrow1,
                   accum, gsem0, gsem1, ssem, asrc_v, adst_v, h_hbm,
                   TPB2 // 2, 0, w * (TPB2 * BLK))
        plsc.subcore_barrier()
        _drain_accum(wrow1, accum, out_hbm, row0, c * NA + row0)

    return k(h2, asrc2, adst2, srcb, dstb)


def kernel(x, edge_index, W1, att_src1, att_dst1, b1, W2, att_src2,
           att_dst2, b2):
    src = edge_index[0]
    dst = edge_index[1]
    srcb = jnp.pad(src, (0, EP - E))
    dstb = jnp.pad(dst, (0, EP - E))
    xp = jnp.pad(x, ((0, NP - N), (0, 0)))

    h_t, asrc, adst = _tc_layer1_pre(
        xp, W1.reshape(F_IN, H1, HID).transpose(1, 0, 2), att_src1,
        att_dst1)
    acc1 = _sc_layer1(h_t, asrc, adst, srcb, dstb)

    h2, asrc2, adst2 = _tc_layer2_pre(acc1, b1, W2, att_src2, att_dst2)

    acc2 = _sc_layer2(h2, asrc2.reshape(NA), adst2.reshape(NA), srcb, dstb)

    return _tc_final(acc2, b2)
